# baseline (speedup 1.0000x reference)
import functools

import jax
import jax.numpy as jnp
from jax import lax
from jax.experimental import pallas as pl
from jax.experimental.pallas import tpu as pltpu
from jax.experimental.pallas import tpu_sc as plsc

_B, _L, _D = 4096, 200, 128
_NC, _NS = 2, 16
_NW = _NC * _NS
_K = 8
_R = 8
_B_STREAM = 3072
_B_SHARED = 768
_B_H2H = _B - _B_STREAM - _B_SHARED
_BPW_S = _B_STREAM // _NW
_BPW_H = _B_SHARED // _NW
_BPW_D = _B_H2H // _NW


@functools.partial(
    pl.kernel,
    mesh=plsc.VectorSubcoreMesh(core_axis_name="c", subcore_axis_name="s"),
    out_type=jax.ShapeDtypeStruct((_B, _L, _D), jnp.float32),
    scratch_types=[
        pltpu.VMEM((_L, _D), jnp.float32),
        pltpu.VMEM_SHARED((_R, _L, _D), jnp.float32),
        pltpu.SemaphoreType.DMA,
        pltpu.SemaphoreType.DMA,
        pltpu.SemaphoreType.DMA,
    ],
)
def _sc_broadcast2(table_hbm, out_hbm, tab_v, tab_sh, sem, sem2, sem3):
    sid = lax.axis_index("s")
    wid = sid * _NC + lax.axis_index("c")
    pltpu.sync_copy(table_hbm.at[pl.ds(0, _L)], tab_v)

    @pl.when(sid == 0)
    def _stage_shared():
        for r in range(_R):
            pltpu.make_async_copy(
                table_hbm.at[pl.ds(0, _L)], tab_sh.at[r], sem2
            ).start()
        for r in range(_R):
            pltpu.make_async_copy(
                table_hbm.at[pl.ds(0, _L)], tab_sh.at[r], sem2
            ).wait()

    plsc.subcore_barrier()

    hbase = _B_STREAM + wid * _BPW_H
    for t in range(_BPW_H // _R):
        pltpu.make_async_copy(
            tab_sh, out_hbm.at[pl.ds(hbase + t * _R, _R)], sem2
        ).start()

    dbase = _B_STREAM + _B_SHARED + wid * _BPW_D
    for t in range(_BPW_D):
        pltpu.make_async_copy(
            table_hbm.at[pl.ds(0, _L)], out_hbm.at[dbase + t], sem3
        ).start()

    base = wid * _BPW_S

    def chunk(j, c):
        row = base + j * _K
        for t in range(_K):
            pltpu.make_async_copy(tab_v, out_hbm.at[row + t], sem).start()
        for t in range(_K):
            pltpu.make_async_copy(tab_v, out_hbm.at[row + t], sem).wait()
        return c

    lax.fori_loop(0, _BPW_S // _K, chunk, 0)

    for t in range(_BPW_H // _R):
        pltpu.make_async_copy(
            tab_sh, out_hbm.at[pl.ds(hbase + t * _R, _R)], sem2
        ).wait()
    for t in range(_BPW_D):
        pltpu.make_async_copy(
            table_hbm.at[pl.ds(0, _L)], out_hbm.at[dbase + t], sem3
        ).wait()


def kernel(sequence, table):
    return _sc_broadcast2(table)


# speedup vs baseline: 5.4469x; 5.4469x over previous
import functools

import jax
import jax.numpy as jnp
from jax import lax
from jax.experimental import pallas as pl
from jax.experimental.pallas import tpu as pltpu
from jax.experimental.pallas import tpu_sc as plsc

_B, _L, _D = 4096, 200, 128
_NC, _NS = 2, 16
_NW = _NC * _NS
_K = 8
_R = 8
_B_STREAM = 3328
_B_SHARED = _B - _B_STREAM
_BPW_S = _B_STREAM // _NW
_BPW_H = _B_SHARED // _NW


@functools.partial(
    pl.kernel,
    mesh=plsc.VectorSubcoreMesh(core_axis_name="c", subcore_axis_name="s"),
    out_type=jax.ShapeDtypeStruct((_B, _L, _D), jnp.float32),
    scratch_types=[
        pltpu.VMEM((_L, _D), jnp.float32),
        pltpu.VMEM_SHARED((_R, _L, _D), jnp.float32),
        pltpu.SemaphoreType.DMA,
        pltpu.SemaphoreType.DMA,
    ],
)
def _sc_broadcast2(table_hbm, out_hbm, tab_v, tab_sh, sem, sem2):
    sid = lax.axis_index("s")
    wid = sid * _NC + lax.axis_index("c")
    pltpu.sync_copy(table_hbm.at[pl.ds(0, _L)], tab_v)

    @pl.when(sid == 0)
    def _stage_shared():
        for r in range(_R):
            pltpu.make_async_copy(
                table_hbm.at[pl.ds(0, _L)], tab_sh.at[r], sem2
            ).start()
        for r in range(_R):
            pltpu.make_async_copy(
                table_hbm.at[pl.ds(0, _L)], tab_sh.at[r], sem2
            ).wait()

    plsc.subcore_barrier()

    hbase = _B_STREAM + wid * _BPW_H
    for t in range(_BPW_H // _R):
        pltpu.make_async_copy(
            tab_sh, out_hbm.at[pl.ds(hbase + t * _R, _R)], sem2
        ).start()

    base = wid * _BPW_S

    def chunk(j, c):
        row = base + j * _K
        for t in range(_K):
            pltpu.make_async_copy(tab_v, out_hbm.at[row + t], sem).start()
        for t in range(_K):
            pltpu.make_async_copy(tab_v, out_hbm.at[row + t], sem).wait()
        return c

    lax.fori_loop(0, _BPW_S // _K, chunk, 0)

    for t in range(_BPW_H // _R):
        pltpu.make_async_copy(
            tab_sh, out_hbm.at[pl.ds(hbase + t * _R, _R)], sem2
        ).wait()


def kernel(sequence, table):
    return _sc_broadcast2(table)
